# baseline (speedup 1.0000x reference)
import jax
import jax.numpy as jnp
from jax.experimental import pallas as pl
from jax.experimental.pallas import tpu as pltpu


def _mlp_kernel(x_ref, w1_ref, b1_ref, w2_ref, b2_ref, o_ref):
    h = jnp.dot(x_ref[0], w1_ref[0], preferred_element_type=jnp.float32)
    h = jnp.maximum(h + b1_ref[0], 0.0)
    o = jnp.dot(h, w2_ref[0], preferred_element_type=jnp.float32)
    o_ref[0] = o + b2_ref[0]


def kernel(x, W1, b1, W2, b2):
    B, S, D = x.shape
    E, _, F = W1.shape
    T = B * S
    per = T // E
    BT = 512
    xr = x.reshape(E, per, D)
    out = pl.pallas_call(
        _mlp_kernel,
        grid=(E, per // BT),
        in_specs=[
            pl.BlockSpec((1, BT, D), lambda e, t: (e, t, 0)),
            pl.BlockSpec((1, D, F), lambda e, t: (e, 0, 0)),
            pl.BlockSpec((1, 1, F), lambda e, t: (e, 0, 0)),
            pl.BlockSpec((1, F, D), lambda e, t: (e, 0, 0)),
            pl.BlockSpec((1, 1, D), lambda e, t: (e, 0, 0)),
        ],
        out_specs=pl.BlockSpec((1, BT, D), lambda e, t: (e, t, 0)),
        out_shape=jax.ShapeDtypeStruct((E, per, D), x.dtype),
        compiler_params=pltpu.CompilerParams(
            dimension_semantics=("arbitrary", "arbitrary"),
        ),
    )(xr, W1, b1.reshape(E, 1, F), W2, b2.reshape(E, 1, D))
    return out.reshape(B, S, D)


# speedup vs baseline: 1.2754x; 1.2754x over previous
import jax
import jax.numpy as jnp
from jax.experimental import pallas as pl
from jax.experimental.pallas import tpu as pltpu


def _mlp_kernel(x_ref, w1_ref, b1_ref, w2_ref, b2_ref, o_ref):
    xb = x_ref[0].astype(jnp.bfloat16)
    w1b = w1_ref[0].astype(jnp.bfloat16)
    h = jnp.dot(xb, w1b, preferred_element_type=jnp.float32)
    h = jnp.maximum(h + b1_ref[0], 0.0).astype(jnp.bfloat16)
    w2b = w2_ref[0].astype(jnp.bfloat16)
    o = jnp.dot(h, w2b, preferred_element_type=jnp.float32)
    o_ref[0] = o + b2_ref[0]


def kernel(x, W1, b1, W2, b2):
    B, S, D = x.shape
    E, _, F = W1.shape
    T = B * S
    per = T // E
    xr = x.reshape(E, per, D)
    out = pl.pallas_call(
        _mlp_kernel,
        grid=(E,),
        in_specs=[
            pl.BlockSpec((1, per, D), lambda e: (e, 0, 0)),
            pl.BlockSpec((1, D, F), lambda e: (e, 0, 0)),
            pl.BlockSpec((1, 1, F), lambda e: (e, 0, 0)),
            pl.BlockSpec((1, F, D), lambda e: (e, 0, 0)),
            pl.BlockSpec((1, 1, D), lambda e: (e, 0, 0)),
        ],
        out_specs=pl.BlockSpec((1, per, D), lambda e: (e, 0, 0)),
        out_shape=jax.ShapeDtypeStruct((E, per, D), x.dtype),
        compiler_params=pltpu.CompilerParams(
            dimension_semantics=("arbitrary",),
            vmem_limit_bytes=112 * 1024 * 1024,
        ),
    )(xr, W1, b1.reshape(E, 1, F), W2, b2.reshape(E, 1, D))
    return out.reshape(B, S, D)
